# Initial kernel scaffold; baseline (speedup 1.0000x reference)
import jax
import jax.numpy as jnp
from jax.experimental import pallas as pl


def kernel(head_emb, tail_emb, rel_idx, relation_embeddings):
    raise NotImplementedError("write your pallas kernel here")


# speedup vs baseline: 1.1052x; 1.1052x over previous
import functools

import jax
import jax.numpy as jnp
from jax import lax
from jax.experimental import pallas as pl
from jax.experimental.pallas import tpu as pltpu
from jax.experimental.pallas import tpu_sc as plsc

NUM_RELATIONS = 1000
D = 64
B = 16384
NC = 2
NS = 16
L = 16
NW = NC * NS
BPW = B // NW

_mesh = plsc.VectorSubcoreMesh(core_axis_name="c", subcore_axis_name="s")


@functools.partial(
    pl.kernel,
    mesh=_mesh,
    out_type=jax.ShapeDtypeStruct((B,), jnp.float32),
    compiler_params=pltpu.CompilerParams(
        needs_layout_passes=False, use_tc_tiling_on_sc=False),
    scratch_types=[
        pltpu.VMEM((BPW,), jnp.int32),
        pltpu.VMEM((BPW, D), jnp.float32),
        pltpu.VMEM((BPW, D), jnp.float32),
        pltpu.VMEM((BPW, D), jnp.float32),
        pltpu.VMEM((L * L,), jnp.float32),
        pltpu.VMEM((BPW,), jnp.float32),
        pltpu.SemaphoreType.DMA,
    ],
)
def _distmult_sc(head_hbm, tail_hbm, idx_hbm, table_hbm, out_hbm,
                 idx_v, rel_v, head_v, tail_v, tbuf, out_v, sem):
    wid = lax.axis_index("s") * NC + lax.axis_index("c")
    base = wid * BPW

    pltpu.sync_copy(idx_hbm.at[pl.ds(base, BPW)], idx_v)
    gather = pltpu.async_copy(table_hbm.at[idx_v], rel_v, sem)
    pltpu.sync_copy(head_hbm.at[pl.ds(base, BPW)], head_v)
    pltpu.sync_copy(tail_hbm.at[pl.ds(base, BPW)], tail_v)
    gather.wait()

    lane_iota_l = lax.iota(jnp.int32, L) * L

    def group_body(g, carry):
        row0 = g * L
        for r in range(L):
            row = row0 + r
            acc = (head_v[row, pl.ds(0, L)] * rel_v[row, pl.ds(0, L)]
                   * tail_v[row, pl.ds(0, L)])
            for q in range(1, D // L):
                acc = acc + (head_v[row, pl.ds(q * L, L)]
                             * rel_v[row, pl.ds(q * L, L)]
                             * tail_v[row, pl.ds(q * L, L)])
            tbuf[pl.ds(r * L, L)] = acc
        sums = plsc.load_gather(tbuf, [lane_iota_l])
        for l in range(1, L):
            sums = sums + plsc.load_gather(tbuf, [lane_iota_l + l])
        out_v[pl.ds(row0, L)] = sums
        return carry

    lax.fori_loop(0, BPW // L, group_body, 0)
    pltpu.sync_copy(out_v, out_hbm.at[pl.ds(base, BPW)])


def kernel(head_emb, tail_emb, rel_idx, relation_embeddings):
    idx = rel_idx.astype(jnp.int32)
    return _distmult_sc(head_emb, tail_emb, idx, relation_embeddings)
